# baseline (speedup 1.0000x reference)
import numpy as np
import jax
import jax.numpy as jnp
from jax.experimental import pallas as pl
from jax.experimental.pallas import tpu as pltpu

_C = 256
_K = 8192
_B = 2
_N = 1024
_SCALES = (1, 2, 4, 8, 16, 32, 64)
_SN = len(_SCALES)
_BN = _B * _N
_TR = 256
_SEGS = ((0, 128), (128, 192), (192, 224), (224, 256))
_PREC_HI = jax.lax.Precision.HIGHEST
_DN = (((1,), (1,)), ((), ()))


def _np_consts():
    def up_idx(pn):
        src = (np.arange(_N, dtype=np.float64) + 0.5) * (pn / _N) - 0.5
        src = np.clip(src, 0.0, pn - 1.0)
        i0 = np.floor(src).astype(np.int64)
        i1 = np.minimum(i0 + 1, pn - 1)
        return i0, i1, (src - i0).astype(np.float32)

    offs, off = [], 0
    for pn in _SCALES:
        offs.append(off)
        off += _B * pn
    rows_pad = 256
    U0 = np.zeros((_BN, rows_pad), np.float32)
    U1 = np.zeros((_BN, rows_pad), np.float32)
    WL = np.zeros((_BN, _SN), np.float32)
    for t, pn in enumerate(_SCALES):
        i0, i1, w = up_idx(pn)
        for b in range(_B):
            rows = b * _N + np.arange(_N)
            U0[rows, offs[t] + b * pn + i0] = 1.0
            U1[rows, offs[t] + b * pn + i1] = 1.0
            WL[rows, t] = w
    segmask = np.zeros((4, _C), np.float32)
    for k, (lo, hi) in enumerate(_SEGS):
        segmask[k, lo:hi] = 1.0
    return offs, U0, U1, WL, segmask


_OFFS, _U0, _U1, _WL, _SEGMASK = _np_consts()


def _dot(a, b, prec):
    return jax.lax.dot_general(a, b, _DN, precision=prec,
                               preferred_element_type=jnp.float32)


def _linear_over_dim1(v):
    acc = v[:, 0, :]
    for c in range(1, v.shape[1]):
        acc = acc + v[:, c, :]
    return acc


def _tree_half(v):
    while v.shape[1] > 1:
        h = v.shape[1] // 2
        v = v[:, :h, :] + v[:, h:, :]
    return v[:, 0, :]


def _adj_halve(v, times):
    for _ in range(times):
        r = v.shape[0]
        t = v.reshape(r // 2, 2, v.shape[1])
        v = t[:, 0, :] + t[:, 1, :]
    return v


def _pool_sums(frest, pn):
    npool = _N // pn
    G = _BN // npool
    C = frest.shape[1]
    if npool == 16:
        return _tree_half(frest.reshape(G, 16, C)).reshape(G, C)
    if npool == 32:
        v = _linear_over_dim1(frest.reshape(G, 4, 8 * C).reshape(G, 4, 8, C))
        return _tree_half(v)
    if npool == 64:
        v = _linear_over_dim1(frest.reshape(G, 8, 8, C))
        return _tree_half(v)
    nv = npool // 128
    v = frest.reshape(G, nv, 128, C)
    acc = v[:, 0, :, :]
    for k in range(1, nv):
        acc = acc + v[:, k, :, :]
    flat = acc.reshape(G * 128, C)
    flat = _adj_halve(flat, 3)
    return _linear_over_dim1(flat.reshape(G, 16, C))


def _seg_sum_rows(x, lo, hi):
    w = hi - lo
    v = x[lo:hi, :]
    M = x.shape[1]
    if w == 128:
        flat = _adj_halve(v, 3)
        return _linear_over_dim1(flat.reshape(1, 16, M)).reshape(1, M)
    nch = w // 8
    vv = v.reshape(nch, 8, M)
    acc = vv[0]
    for c in range(1, nch):
        acc = acc + vv[c]
    return _tree_half(acc.reshape(1, 8, M)).reshape(1, M)


def _prep_body(f2_ref, base_ref, wp_ref, sm_ref, emb_ref, ee4_ref, eef_ref):
    emb = _dot(base_ref[...], wp_ref[...], None)
    emb_ref[...] = emb
    ident = (jax.lax.broadcasted_iota(jnp.int32, (_C, _C), 0) ==
             jax.lax.broadcasted_iota(jnp.int32, (_C, _C), 1)).astype(jnp.float32)
    emb2T = _dot(ident, emb * emb, _PREC_HI)
    for k, (lo, hi) in enumerate(_SEGS):
        ee4_ref[k:k + 1, :] = _seg_sum_rows(emb2T, lo, hi)
    f2sq = f2_ref[...] * f2_ref[...]
    sm = sm_ref[...]
    for k in range(4):
        eef_ref[k:k + 1, :] = _dot(sm[k:k + 1, :], f2sq, _PREC_HI)


def _vq_core_body(f2_ref, emb_ref, ee4_ref, u0_ref, u1_ref, wl_ref, sm_ref,
                  fout_ref, fhall_ref, frest_ref, fhat_ref, d_ref, sem):
    f2 = f2_ref[...]
    sm = sm_ref[...]
    frest_ref[...] = f2
    fhat_ref[...] = jnp.zeros((_BN, _C), jnp.float32)
    for t, pn in enumerate(_SCALES):
        npool = _N // pn
        rr = _B * pn
        off = _OFFS[t]
        rest = _pool_sums(frest_ref[...], pn) * jnp.float32(1.0 / npool)
        rsq = rest * rest
        for k, (lo, hi) in enumerate(_SEGS):
            qq = _dot(rsq, sm[k:k + 1, :], _PREC_HI)
            qe = _dot(rest[:, lo:hi], emb_ref[:, lo:hi], None)
            dk = (qq + ee4_ref[k:k + 1, :] - 2.0 * qe) / jnp.float32(hi - lo)
            if k == 0:
                d_ref[0:rr, :] = dk
            else:
                d_ref[0:rr, :] = d_ref[0:rr, :] + dk
        d = d_ref[0:rr, :]
        dmin = jnp.min(d, axis=1, keepdims=True)
        iota = jax.lax.broadcasted_iota(jnp.int32, (rr, _K), 1)
        idx = jnp.min(jnp.where(d <= dmin, iota, _K), axis=1, keepdims=True)
        oh = (iota == idx).astype(jnp.float32)
        H = jnp.dot(oh, emb_ref[...], precision=_PREC_HI,
                    preferred_element_type=jnp.float32)
        h0 = jnp.dot(u0_ref[:, off:off + rr], H, precision=_PREC_HI,
                     preferred_element_type=jnp.float32)
        h1 = jnp.dot(u1_ref[:, off:off + rr], H, precision=_PREC_HI,
                     preferred_element_type=jnp.float32)
        w = wl_ref[:, t:t + 1]
        up = h0 * (1.0 - w) + h1 * w
        fhat_ref[...] = fhat_ref[...] + up
        frest_ref[...] = frest_ref[...] - up
        cp = pltpu.make_async_copy(
            fhat_ref, fhall_ref.at[pl.ds(t * _BN, _BN), :], sem)
        cp.start()
        cp.wait()
    fout_ref[...] = (fhat_ref[...] - f2) + f2


def _loss_body(fhall_ref, f2_ref, sm_ref, eef_ref, lat_ref, com_ref, acc_ref):
    s = pl.program_id(1)
    fh = fhall_ref[...]
    f2 = f2_ref[...]
    sm = sm_ref[...]
    fsq = fh * fh
    d = None
    for k, (lo, hi) in enumerate(_SEGS):
        qq = _dot(fsq, sm[k:k + 1, :], _PREC_HI)
        qe = _dot(fh[:, lo:hi], f2[:, lo:hi], None)
        dk = (qq + eef_ref[k:k + 1, :] - 2.0 * qe) / jnp.float32(hi - lo)
        d = dk if d is None else d + dk

    @pl.when(s == 0)
    def _():
        acc_ref[...] = d

    @pl.when(s > 0)
    def _():
        acc_ref[...] = acc_ref[...] + d

    @pl.when(s == _SN - 1)
    def _():
        m = acc_ref[...] * jnp.float32(1.0 / _SN)
        lat_ref[...] = m
        com_ref[...] = 0.25 * m


def kernel(f_BNC, base, W_proj):
    f2 = f_BNC.reshape(_BN, _C)
    u0 = jnp.asarray(_U0)
    u1 = jnp.asarray(_U1)
    wl = jnp.asarray(_WL)
    sm = jnp.asarray(_SEGMASK)

    emb, ee4, eef = pl.pallas_call(
        _prep_body,
        out_shape=(
            jax.ShapeDtypeStruct((_K, _C), jnp.float32),
            jax.ShapeDtypeStruct((4, _K), jnp.float32),
            jax.ShapeDtypeStruct((4, _BN), jnp.float32),
        ),
    )(f2, base, W_proj, sm)

    fout, fhall = pl.pallas_call(
        _vq_core_body,
        out_shape=(
            jax.ShapeDtypeStruct((_BN, _C), jnp.float32),
            jax.ShapeDtypeStruct((_SN * _BN, _C), jnp.float32),
        ),
        out_specs=(
            pl.BlockSpec(memory_space=pltpu.VMEM),
            pl.BlockSpec(memory_space=pl.ANY),
        ),
        scratch_shapes=[
            pltpu.VMEM((_BN, _C), jnp.float32),
            pltpu.VMEM((_BN, _C), jnp.float32),
            pltpu.VMEM((_B * 64, _K), jnp.float32),
            pltpu.SemaphoreType.DMA,
        ],
    )(f2, emb, ee4, u0, u1, wl, sm)

    nrt = _BN // _TR
    lat, com = pl.pallas_call(
        _loss_body,
        grid=(nrt, _SN),
        in_specs=[
            pl.BlockSpec((_TR, _C), lambda i, s: (s * nrt + i, 0)),
            pl.BlockSpec((_BN, _C), lambda i, s: (0, 0)),
            pl.BlockSpec((4, _C), lambda i, s: (0, 0)),
            pl.BlockSpec((4, _BN), lambda i, s: (0, 0)),
        ],
        out_specs=[
            pl.BlockSpec((_TR, _BN), lambda i, s: (i, 0)),
            pl.BlockSpec((_TR, _BN), lambda i, s: (i, 0)),
        ],
        out_shape=(
            jax.ShapeDtypeStruct((_BN, _BN), jnp.float32),
            jax.ShapeDtypeStruct((_BN, _BN), jnp.float32),
        ),
        scratch_shapes=[pltpu.VMEM((_TR, _BN), jnp.float32)],
    )(fhall, f2, sm, eef)

    return (fout.reshape(_B, _N, _C), com, lat)


# speedup vs baseline: 1.2925x; 1.2925x over previous
import numpy as np
import jax
import jax.numpy as jnp
from jax.experimental import pallas as pl
from jax.experimental.pallas import tpu as pltpu

_C = 256
_K = 8192
_B = 2
_N = 1024
_SCALES = (1, 2, 4, 8, 16, 32, 64)
_SN = len(_SCALES)
_BN = _B * _N
_TR = 256
_SEGS = ((0, 128), (128, 192), (192, 224), (224, 256))
_PREC_HI = jax.lax.Precision.HIGHEST
_DN = (((1,), (1,)), ((), ()))


def _np_consts():
    def up_idx(pn):
        src = (np.arange(_N, dtype=np.float64) + 0.5) * (pn / _N) - 0.5
        src = np.clip(src, 0.0, pn - 1.0)
        i0 = np.floor(src).astype(np.int64)
        i1 = np.minimum(i0 + 1, pn - 1)
        return i0, i1, (src - i0).astype(np.float32)

    offs, off = [], 0
    for pn in _SCALES:
        offs.append(off)
        off += _B * pn
    rows_pad = 256
    U0 = np.zeros((_BN, rows_pad), np.float32)
    U1 = np.zeros((_BN, rows_pad), np.float32)
    WL = np.zeros((_BN, _SN), np.float32)
    for t, pn in enumerate(_SCALES):
        i0, i1, w = up_idx(pn)
        for b in range(_B):
            rows = b * _N + np.arange(_N)
            U0[rows, offs[t] + b * pn + i0] = 1.0
            U1[rows, offs[t] + b * pn + i1] = 1.0
            WL[rows, t] = w
    segmask = np.zeros((4, _C), np.float32)
    wch = np.zeros((1, _C), np.float32)
    for k, (lo, hi) in enumerate(_SEGS):
        segmask[k, lo:hi] = 1.0
        wch[0, lo:hi] = 1.0 / (hi - lo)
    return offs, U0, U1, WL, segmask, wch


_OFFS, _U0, _U1, _WL, _SEGMASK, _WCH = _np_consts()


def _dot(a, b, prec):
    return jax.lax.dot_general(a, b, _DN, precision=prec,
                               preferred_element_type=jnp.float32)


def _linear_over_dim1(v):
    acc = v[:, 0, :]
    for c in range(1, v.shape[1]):
        acc = acc + v[:, c, :]
    return acc


def _tree_half(v):
    while v.shape[1] > 1:
        h = v.shape[1] // 2
        v = v[:, :h, :] + v[:, h:, :]
    return v[:, 0, :]


def _adj_halve(v, times):
    for _ in range(times):
        r = v.shape[0]
        t = v.reshape(r // 2, 2, v.shape[1])
        v = t[:, 0, :] + t[:, 1, :]
    return v


def _pool_sums(frest, pn):
    npool = _N // pn
    G = _BN // npool
    C = frest.shape[1]
    if npool == 16:
        return _tree_half(frest.reshape(G, 16, C)).reshape(G, C)
    if npool == 32:
        v = _linear_over_dim1(frest.reshape(G, 4, 8 * C).reshape(G, 4, 8, C))
        return _tree_half(v)
    if npool == 64:
        v = _linear_over_dim1(frest.reshape(G, 8, 8, C))
        return _tree_half(v)
    nv = npool // 128
    v = frest.reshape(G, nv, 128, C)
    acc = v[:, 0, :, :]
    for k in range(1, nv):
        acc = acc + v[:, k, :, :]
    flat = acc.reshape(G * 128, C)
    flat = _adj_halve(flat, 3)
    return _linear_over_dim1(flat.reshape(G, 16, C))


def _seg_sum_rows(x, lo, hi):
    w = hi - lo
    v = x[lo:hi, :]
    M = x.shape[1]
    if w == 128:
        flat = _adj_halve(v, 3)
        return _linear_over_dim1(flat.reshape(1, 16, M)).reshape(1, M)
    nch = w // 8
    vv = v.reshape(nch, 8, M)
    acc = vv[0]
    for c in range(1, nch):
        acc = acc + vv[c]
    return _tree_half(acc.reshape(1, 8, M)).reshape(1, M)


def _prep_body(f2_ref, base_ref, wp_ref, sm_ref, wch_ref, emb_ref, ee4_ref,
               eew_ref):
    emb = _dot(base_ref[...], wp_ref[...], None)
    emb_ref[...] = emb
    ident = (jax.lax.broadcasted_iota(jnp.int32, (_C, _C), 0) ==
             jax.lax.broadcasted_iota(jnp.int32, (_C, _C), 1)).astype(jnp.float32)
    emb2T = _dot(ident, emb * emb, _PREC_HI)
    for k, (lo, hi) in enumerate(_SEGS):
        ee4_ref[k:k + 1, :] = _seg_sum_rows(emb2T, lo, hi)
    f2sq = f2_ref[...] * f2_ref[...]
    eew_ref[...] = _dot(wch_ref[...], f2sq, _PREC_HI)


def _vq_core_body(f2_ref, emb_ref, ee4_ref, u0_ref, u1_ref, wl_ref, sm_ref,
                  fout_ref, fhall_ref, frest_ref, fhat_ref, d_ref, sem):
    f2 = f2_ref[...]
    sm = sm_ref[...]
    frest_ref[...] = f2
    fhat_ref[...] = jnp.zeros((_BN, _C), jnp.float32)
    for t, pn in enumerate(_SCALES):
        npool = _N // pn
        rr = _B * pn
        off = _OFFS[t]
        rest = _pool_sums(frest_ref[...], pn) * jnp.float32(1.0 / npool)
        rsq = rest * rest
        for k, (lo, hi) in enumerate(_SEGS):
            qq = _dot(rsq, sm[k:k + 1, :], _PREC_HI)
            qe = _dot(rest[:, lo:hi], emb_ref[:, lo:hi], None)
            dk = (qq + ee4_ref[k:k + 1, :] - 2.0 * qe) / jnp.float32(hi - lo)
            if k == 0:
                d_ref[0:rr, :] = dk
            else:
                d_ref[0:rr, :] = d_ref[0:rr, :] + dk
        d = d_ref[0:rr, :]
        dmin = jnp.min(d, axis=1, keepdims=True)
        iota = jax.lax.broadcasted_iota(jnp.int32, (rr, _K), 1)
        idx = jnp.min(jnp.where(d <= dmin, iota, _K), axis=1, keepdims=True)
        oh = (iota == idx).astype(jnp.float32)
        H = jnp.dot(oh, emb_ref[...], precision=_PREC_HI,
                    preferred_element_type=jnp.float32)
        h0 = jnp.dot(u0_ref[:, off:off + rr], H, precision=_PREC_HI,
                     preferred_element_type=jnp.float32)
        h1 = jnp.dot(u1_ref[:, off:off + rr], H, precision=_PREC_HI,
                     preferred_element_type=jnp.float32)
        w = wl_ref[:, t:t + 1]
        up = h0 * (1.0 - w) + h1 * w
        fhat_ref[...] = fhat_ref[...] + up
        frest_ref[...] = frest_ref[...] - up
        cp = pltpu.make_async_copy(
            fhat_ref, fhall_ref.at[pl.ds(t * _BN, _BN), :], sem)
        cp.start()
        cp.wait()
    fout_ref[...] = (fhat_ref[...] - f2) + f2


def _loss_body(fhall_ref, f2_ref, wch_ref, eew_ref, lat_ref, com_ref, acc_ref):
    s = pl.program_id(1)
    fh = fhall_ref[...]
    wch = wch_ref[...]
    qq = _dot(fh * fh, wch, _PREC_HI)
    qe = _dot(fh * wch, f2_ref[...], None)
    d = (qq + eew_ref[...]) - 2.0 * qe

    @pl.when(s == 0)
    def _():
        acc_ref[...] = d

    @pl.when(s > 0)
    def _():
        acc_ref[...] = acc_ref[...] + d

    @pl.when(s == _SN - 1)
    def _():
        m = acc_ref[...] * jnp.float32(1.0 / _SN)
        lat_ref[...] = m
        com_ref[...] = 0.25 * m


def kernel(f_BNC, base, W_proj):
    f2 = f_BNC.reshape(_BN, _C)
    u0 = jnp.asarray(_U0)
    u1 = jnp.asarray(_U1)
    wl = jnp.asarray(_WL)
    sm = jnp.asarray(_SEGMASK)

    wch = jnp.asarray(_WCH)
    emb, ee4, eew = pl.pallas_call(
        _prep_body,
        out_shape=(
            jax.ShapeDtypeStruct((_K, _C), jnp.float32),
            jax.ShapeDtypeStruct((4, _K), jnp.float32),
            jax.ShapeDtypeStruct((1, _BN), jnp.float32),
        ),
    )(f2, base, W_proj, sm, wch)

    fout, fhall = pl.pallas_call(
        _vq_core_body,
        out_shape=(
            jax.ShapeDtypeStruct((_BN, _C), jnp.float32),
            jax.ShapeDtypeStruct((_SN * _BN, _C), jnp.float32),
        ),
        out_specs=(
            pl.BlockSpec(memory_space=pltpu.VMEM),
            pl.BlockSpec(memory_space=pl.ANY),
        ),
        scratch_shapes=[
            pltpu.VMEM((_BN, _C), jnp.float32),
            pltpu.VMEM((_BN, _C), jnp.float32),
            pltpu.VMEM((_B * 64, _K), jnp.float32),
            pltpu.SemaphoreType.DMA,
        ],
    )(f2, emb, ee4, u0, u1, wl, sm)

    nrt = _BN // _TR
    lat, com = pl.pallas_call(
        _loss_body,
        grid=(nrt, _SN),
        in_specs=[
            pl.BlockSpec((_TR, _C), lambda i, s: (s * nrt + i, 0)),
            pl.BlockSpec((_BN, _C), lambda i, s: (0, 0)),
            pl.BlockSpec((1, _C), lambda i, s: (0, 0)),
            pl.BlockSpec((1, _BN), lambda i, s: (0, 0)),
        ],
        out_specs=[
            pl.BlockSpec((_TR, _BN), lambda i, s: (i, 0)),
            pl.BlockSpec((_TR, _BN), lambda i, s: (i, 0)),
        ],
        out_shape=(
            jax.ShapeDtypeStruct((_BN, _BN), jnp.float32),
            jax.ShapeDtypeStruct((_BN, _BN), jnp.float32),
        ),
        scratch_shapes=[pltpu.VMEM((_TR, _BN), jnp.float32)],
    )(fhall, f2, wch, eew)

    return (fout.reshape(_B, _N, _C), com, lat)


# speedup vs baseline: 1.3673x; 1.0578x over previous
import numpy as np
import jax
import jax.numpy as jnp
from jax.experimental import pallas as pl
from jax.experimental.pallas import tpu as pltpu

_C = 256
_K = 8192
_B = 2
_N = 1024
_SCALES = (1, 2, 4, 8, 16, 32, 64)
_SN = len(_SCALES)
_BN = _B * _N
_TR = 256
_SEGS = ((0, 128), (128, 192), (192, 224), (224, 256))
_PREC_HI = jax.lax.Precision.HIGHEST
_DN = (((1,), (1,)), ((), ()))


def _np_consts():
    def up_idx(pn):
        src = (np.arange(_N, dtype=np.float64) + 0.5) * (pn / _N) - 0.5
        src = np.clip(src, 0.0, pn - 1.0)
        i0 = np.floor(src).astype(np.int64)
        i1 = np.minimum(i0 + 1, pn - 1)
        return i0, i1, (src - i0).astype(np.float32)

    offs, off = [], 0
    for pn in _SCALES:
        offs.append(off)
        off += _B * pn
    rows_pad = 256
    U0 = np.zeros((_BN, rows_pad), np.float32)
    U1 = np.zeros((_BN, rows_pad), np.float32)
    WL = np.zeros((_BN, _SN), np.float32)
    for t, pn in enumerate(_SCALES):
        i0, i1, w = up_idx(pn)
        for b in range(_B):
            rows = b * _N + np.arange(_N)
            U0[rows, offs[t] + b * pn + i0] = 1.0
            U1[rows, offs[t] + b * pn + i1] = 1.0
            WL[rows, t] = w
    segmask = np.zeros((4, _C), np.float32)
    wch = np.zeros((1, _C), np.float32)
    for k, (lo, hi) in enumerate(_SEGS):
        segmask[k, lo:hi] = 1.0
        wch[0, lo:hi] = 1.0 / (hi - lo)
    return offs, U0, U1, WL, segmask, wch


_OFFS, _U0, _U1, _WL, _SEGMASK, _WCH = _np_consts()


def _dot(a, b, prec):
    return jax.lax.dot_general(a, b, _DN, precision=prec,
                               preferred_element_type=jnp.float32)


def _linear_over_dim1(v):
    acc = v[:, 0, :]
    for c in range(1, v.shape[1]):
        acc = acc + v[:, c, :]
    return acc


def _tree_half(v):
    while v.shape[1] > 1:
        h = v.shape[1] // 2
        v = v[:, :h, :] + v[:, h:, :]
    return v[:, 0, :]


def _adj_halve(v, times):
    for _ in range(times):
        r = v.shape[0]
        t = v.reshape(r // 2, 2, v.shape[1])
        v = t[:, 0, :] + t[:, 1, :]
    return v


def _pool_sums(frest, pn):
    npool = _N // pn
    G = _BN // npool
    C = frest.shape[1]
    if npool == 16:
        return _tree_half(frest.reshape(G, 16, C)).reshape(G, C)
    if npool == 32:
        v = _linear_over_dim1(frest.reshape(G, 4, 8 * C).reshape(G, 4, 8, C))
        return _tree_half(v)
    if npool == 64:
        v = _linear_over_dim1(frest.reshape(G, 8, 8, C))
        return _tree_half(v)
    nv = npool // 128
    v = frest.reshape(G, nv, 128, C)
    acc = v[:, 0, :, :]
    for k in range(1, nv):
        acc = acc + v[:, k, :, :]
    flat = acc.reshape(G * 128, C)
    flat = _adj_halve(flat, 3)
    return _linear_over_dim1(flat.reshape(G, 16, C))


def _seg_sum_rows(x, lo, hi):
    w = hi - lo
    v = x[lo:hi, :]
    M = x.shape[1]
    if w == 128:
        flat = _adj_halve(v, 3)
        return _linear_over_dim1(flat.reshape(1, 16, M)).reshape(1, M)
    nch = w // 8
    vv = v.reshape(nch, 8, M)
    acc = vv[0]
    for c in range(1, nch):
        acc = acc + vv[c]
    return _tree_half(acc.reshape(1, 8, M)).reshape(1, M)


def _prep_body(f2_ref, base_ref, wp_ref, sm_ref, wch_ref, emb_ref, ee4_ref,
               eew_ref):
    emb = _dot(base_ref[...], wp_ref[...], None)
    emb_ref[...] = emb
    emb2T = jnp.transpose(emb * emb, (1, 0))
    for k, (lo, hi) in enumerate(_SEGS):
        ee4_ref[k:k + 1, :] = _seg_sum_rows(emb2T, lo, hi)
    f2sq = f2_ref[...] * f2_ref[...]
    eew_ref[...] = _dot(wch_ref[...], f2sq, _PREC_HI)


def _vq_core_body(f2_ref, emb_ref, ee4_ref, u0_ref, u1_ref, wl_ref, sm_ref,
                  fout_ref, fhall_ref, frest_ref, fhat_ref, d_ref, sem):
    f2 = f2_ref[...]
    sm = sm_ref[...]
    frest_ref[...] = f2
    fhat_ref[...] = jnp.zeros((_BN, _C), jnp.float32)
    for t, pn in enumerate(_SCALES):
        npool = _N // pn
        rr = _B * pn
        off = _OFFS[t]
        rest = _pool_sums(frest_ref[...], pn) * jnp.float32(1.0 / npool)
        qq4 = _dot(rest * rest, sm, _PREC_HI)
        for k, (lo, hi) in enumerate(_SEGS):
            qq = qq4[:, k:k + 1]
            qe = _dot(rest[:, lo:hi], emb_ref[:, lo:hi], None)
            dk = (qq + ee4_ref[k:k + 1, :] - 2.0 * qe) / jnp.float32(hi - lo)
            if k == 0:
                d_ref[0:rr, :] = dk
            else:
                d_ref[0:rr, :] = d_ref[0:rr, :] + dk
        d = d_ref[0:rr, :]
        dmin = jnp.min(d, axis=1, keepdims=True)
        iota = jax.lax.broadcasted_iota(jnp.int32, (rr, _K), 1)
        idx = jnp.min(jnp.where(d <= dmin, iota, _K), axis=1, keepdims=True)
        oh = (iota == idx).astype(jnp.float32)
        H = jnp.dot(oh, emb_ref[...], precision=_PREC_HI,
                    preferred_element_type=jnp.float32)
        h0 = jnp.dot(u0_ref[:, off:off + rr], H, precision=_PREC_HI,
                     preferred_element_type=jnp.float32)
        h1 = jnp.dot(u1_ref[:, off:off + rr], H, precision=_PREC_HI,
                     preferred_element_type=jnp.float32)
        w = wl_ref[:, t:t + 1]
        up = h0 * (1.0 - w) + h1 * w
        fhat_ref[...] = fhat_ref[...] + up
        frest_ref[...] = frest_ref[...] - up
        cp = pltpu.make_async_copy(
            fhat_ref, fhall_ref.at[pl.ds(t * _BN, _BN), :], sem)
        cp.start()
        cp.wait()
    fout_ref[...] = (fhat_ref[...] - f2) + f2


def _loss_body(fhall_ref, f2_ref, wch_ref, eew_ref, lat_ref, com_ref, acc_ref):
    s = pl.program_id(1)
    fh = fhall_ref[...]
    wch = wch_ref[...]
    qq = _dot(fh * fh, wch, _PREC_HI)
    qe = _dot(fh * wch, f2_ref[...], None)
    d = (qq + eew_ref[...]) - 2.0 * qe

    @pl.when(s == 0)
    def _():
        acc_ref[...] = d

    @pl.when(s > 0)
    def _():
        acc_ref[...] = acc_ref[...] + d

    @pl.when(s == _SN - 1)
    def _():
        m = acc_ref[...] * jnp.float32(1.0 / _SN)
        lat_ref[...] = m
        com_ref[...] = 0.25 * m


def kernel(f_BNC, base, W_proj):
    f2 = f_BNC.reshape(_BN, _C)
    u0 = jnp.asarray(_U0)
    u1 = jnp.asarray(_U1)
    wl = jnp.asarray(_WL)
    sm = jnp.asarray(_SEGMASK)

    wch = jnp.asarray(_WCH)
    emb, ee4, eew = pl.pallas_call(
        _prep_body,
        out_shape=(
            jax.ShapeDtypeStruct((_K, _C), jnp.float32),
            jax.ShapeDtypeStruct((4, _K), jnp.float32),
            jax.ShapeDtypeStruct((1, _BN), jnp.float32),
        ),
    )(f2, base, W_proj, sm, wch)

    fout, fhall = pl.pallas_call(
        _vq_core_body,
        out_shape=(
            jax.ShapeDtypeStruct((_BN, _C), jnp.float32),
            jax.ShapeDtypeStruct((_SN * _BN, _C), jnp.float32),
        ),
        out_specs=(
            pl.BlockSpec(memory_space=pltpu.VMEM),
            pl.BlockSpec(memory_space=pl.ANY),
        ),
        scratch_shapes=[
            pltpu.VMEM((_BN, _C), jnp.float32),
            pltpu.VMEM((_BN, _C), jnp.float32),
            pltpu.VMEM((_B * 64, _K), jnp.float32),
            pltpu.SemaphoreType.DMA,
        ],
    )(f2, emb, ee4, u0, u1, wl, sm)

    nrt = _BN // _TR
    lat, com = pl.pallas_call(
        _loss_body,
        grid=(nrt, _SN),
        in_specs=[
            pl.BlockSpec((_TR, _C), lambda i, s: (s * nrt + i, 0)),
            pl.BlockSpec((_BN, _C), lambda i, s: (0, 0)),
            pl.BlockSpec((1, _C), lambda i, s: (0, 0)),
            pl.BlockSpec((1, _BN), lambda i, s: (0, 0)),
        ],
        out_specs=[
            pl.BlockSpec((_TR, _BN), lambda i, s: (i, 0)),
            pl.BlockSpec((_TR, _BN), lambda i, s: (i, 0)),
        ],
        out_shape=(
            jax.ShapeDtypeStruct((_BN, _BN), jnp.float32),
            jax.ShapeDtypeStruct((_BN, _BN), jnp.float32),
        ),
        scratch_shapes=[pltpu.VMEM((_TR, _BN), jnp.float32)],
    )(fhall, f2, wch, eew)

    return (fout.reshape(_B, _N, _C), com, lat)


# speedup vs baseline: 1.4665x; 1.0726x over previous
import numpy as np
import jax
import jax.numpy as jnp
from jax.experimental import pallas as pl
from jax.experimental.pallas import tpu as pltpu

_C = 256
_K = 8192
_B = 2
_N = 1024
_SCALES = (1, 2, 4, 8, 16, 32, 64)
_SN = len(_SCALES)
_BN = _B * _N
_TR = 512
_SEGS = ((0, 128), (128, 192), (192, 224), (224, 256))
_PREC_HI = jax.lax.Precision.HIGHEST
_DN = (((1,), (1,)), ((), ()))


def _np_consts():
    def up_idx(pn):
        src = (np.arange(_N, dtype=np.float64) + 0.5) * (pn / _N) - 0.5
        src = np.clip(src, 0.0, pn - 1.0)
        i0 = np.floor(src).astype(np.int64)
        i1 = np.minimum(i0 + 1, pn - 1)
        return i0, i1, (src - i0).astype(np.float32)

    offs, off = [], 0
    for pn in _SCALES:
        offs.append(off)
        off += _B * pn
    rows_pad = 256
    U0 = np.zeros((_BN, rows_pad), np.float32)
    U1 = np.zeros((_BN, rows_pad), np.float32)
    WL = np.zeros((_BN, _SN), np.float32)
    for t, pn in enumerate(_SCALES):
        i0, i1, w = up_idx(pn)
        for b in range(_B):
            rows = b * _N + np.arange(_N)
            U0[rows, offs[t] + b * pn + i0] = 1.0
            U1[rows, offs[t] + b * pn + i1] = 1.0
            WL[rows, t] = w
    segmask = np.zeros((4, _C), np.float32)
    wch = np.zeros((1, _C), np.float32)
    for k, (lo, hi) in enumerate(_SEGS):
        segmask[k, lo:hi] = 1.0
        wch[0, lo:hi] = 1.0 / (hi - lo)
    return offs, U0, U1, WL, segmask, wch


_OFFS, _U0, _U1, _WL, _SEGMASK, _WCH = _np_consts()


def _dot(a, b, prec):
    return jax.lax.dot_general(a, b, _DN, precision=prec,
                               preferred_element_type=jnp.float32)


def _linear_over_dim1(v):
    acc = v[:, 0, :]
    for c in range(1, v.shape[1]):
        acc = acc + v[:, c, :]
    return acc


def _tree_half(v):
    while v.shape[1] > 1:
        h = v.shape[1] // 2
        v = v[:, :h, :] + v[:, h:, :]
    return v[:, 0, :]


def _adj_halve(v, times):
    for _ in range(times):
        r = v.shape[0]
        t = v.reshape(r // 2, 2, v.shape[1])
        v = t[:, 0, :] + t[:, 1, :]
    return v


def _pool_sums(frest, pn):
    npool = _N // pn
    G = _BN // npool
    C = frest.shape[1]
    if npool == 16:
        return _tree_half(frest.reshape(G, 16, C)).reshape(G, C)
    if npool == 32:
        v = _linear_over_dim1(frest.reshape(G, 4, 8 * C).reshape(G, 4, 8, C))
        return _tree_half(v)
    if npool == 64:
        v = _linear_over_dim1(frest.reshape(G, 8, 8, C))
        return _tree_half(v)
    nv = npool // 128
    v = frest.reshape(G, nv, 128, C)
    acc = v[:, 0, :, :]
    for k in range(1, nv):
        acc = acc + v[:, k, :, :]
    flat = acc.reshape(G * 128, C)
    flat = _adj_halve(flat, 3)
    return _linear_over_dim1(flat.reshape(G, 16, C))


def _seg_sum_rows(x, lo, hi):
    w = hi - lo
    v = x[lo:hi, :]
    M = x.shape[1]
    if w == 128:
        flat = _adj_halve(v, 3)
        return _linear_over_dim1(flat.reshape(1, 16, M)).reshape(1, M)
    nch = w // 8
    vv = v.reshape(nch, 8, M)
    acc = vv[0]
    for c in range(1, nch):
        acc = acc + vv[c]
    return _tree_half(acc.reshape(1, 8, M)).reshape(1, M)


def _prep_body(f2_ref, base_ref, wp_ref, sm_ref, wch_ref, emb_ref, ee4_ref,
               eew_ref):
    emb = _dot(base_ref[...], wp_ref[...], None)
    emb_ref[...] = emb
    emb2T = jnp.transpose(emb * emb, (1, 0))
    for k, (lo, hi) in enumerate(_SEGS):
        ee4_ref[k:k + 1, :] = _seg_sum_rows(emb2T, lo, hi)
    f2sq = f2_ref[...] * f2_ref[...]
    eew_ref[...] = _dot(wch_ref[...], f2sq, _PREC_HI)


def _vq_core_body(f2_ref, emb_ref, ee4_ref, u0_ref, u1_ref, wl_ref, sm_ref,
                  fout_ref, fhall_ref, frest_ref, fhat_ref, sem):
    f2 = f2_ref[...]
    sm = sm_ref[...]
    frest_ref[...] = f2
    fhat_ref[...] = jnp.zeros((_BN, _C), jnp.float32)
    for t, pn in enumerate(_SCALES):
        npool = _N // pn
        rr = _B * pn
        off = _OFFS[t]
        rest = _pool_sums(frest_ref[...], pn) * jnp.float32(1.0 / npool)
        qq4 = _dot(rest * rest, sm, _PREC_HI)
        d = None
        for k, (lo, hi) in enumerate(_SEGS):
            qq = qq4[:, k:k + 1]
            qe = _dot(rest[:, lo:hi], emb_ref[:, lo:hi], None)
            dk = (qq + ee4_ref[k:k + 1, :] - 2.0 * qe) / jnp.float32(hi - lo)
            d = dk if d is None else d + dk
        dmin = jnp.min(d, axis=1, keepdims=True)
        iota = jax.lax.broadcasted_iota(jnp.int32, (rr, _K), 1)
        idx = jnp.min(jnp.where(d <= dmin, iota, _K), axis=1, keepdims=True)
        oh = (iota == idx).astype(jnp.float32)
        H = jnp.dot(oh, emb_ref[...], precision=_PREC_HI,
                    preferred_element_type=jnp.float32)
        h0 = jnp.dot(u0_ref[:, off:off + rr], H, precision=_PREC_HI,
                     preferred_element_type=jnp.float32)
        h1 = jnp.dot(u1_ref[:, off:off + rr], H, precision=_PREC_HI,
                     preferred_element_type=jnp.float32)
        w = wl_ref[:, t:t + 1]
        up = h0 * (1.0 - w) + h1 * w
        fhat_ref[...] = fhat_ref[...] + up
        frest_ref[...] = frest_ref[...] - up
        cp = pltpu.make_async_copy(
            fhat_ref, fhall_ref.at[pl.ds(t * _BN, _BN), :], sem)
        cp.start()
        cp.wait()
    fout_ref[...] = (fhat_ref[...] - f2) + f2


def _loss_body(fhall_ref, f2_ref, wch_ref, eew_ref, lat_ref, com_ref, acc_ref):
    s = pl.program_id(1)
    fh = fhall_ref[...]
    wch = wch_ref[...]
    qq = _dot(fh * fh, wch, _PREC_HI)
    qe = _dot(fh * wch, f2_ref[...], None)
    d = (qq + eew_ref[...]) - 2.0 * qe

    @pl.when(s == 0)
    def _():
        acc_ref[...] = d

    @pl.when(s > 0)
    def _():
        acc_ref[...] = acc_ref[...] + d

    @pl.when(s == _SN - 1)
    def _():
        m = acc_ref[...] * jnp.float32(1.0 / _SN)
        lat_ref[...] = m
        com_ref[...] = 0.25 * m


def kernel(f_BNC, base, W_proj):
    f2 = f_BNC.reshape(_BN, _C)
    u0 = jnp.asarray(_U0)
    u1 = jnp.asarray(_U1)
    wl = jnp.asarray(_WL)
    sm = jnp.asarray(_SEGMASK)

    wch = jnp.asarray(_WCH)
    emb, ee4, eew = pl.pallas_call(
        _prep_body,
        out_shape=(
            jax.ShapeDtypeStruct((_K, _C), jnp.float32),
            jax.ShapeDtypeStruct((4, _K), jnp.float32),
            jax.ShapeDtypeStruct((1, _BN), jnp.float32),
        ),
    )(f2, base, W_proj, sm, wch)

    fout, fhall = pl.pallas_call(
        _vq_core_body,
        out_shape=(
            jax.ShapeDtypeStruct((_BN, _C), jnp.float32),
            jax.ShapeDtypeStruct((_SN * _BN, _C), jnp.float32),
        ),
        out_specs=(
            pl.BlockSpec(memory_space=pltpu.VMEM),
            pl.BlockSpec(memory_space=pl.ANY),
        ),
        scratch_shapes=[
            pltpu.VMEM((_BN, _C), jnp.float32),
            pltpu.VMEM((_BN, _C), jnp.float32),
            pltpu.SemaphoreType.DMA,
        ],
    )(f2, emb, ee4, u0, u1, wl, sm)

    nrt = _BN // _TR
    lat, com = pl.pallas_call(
        _loss_body,
        grid=(nrt, _SN),
        in_specs=[
            pl.BlockSpec((_TR, _C), lambda i, s: (s * nrt + i, 0)),
            pl.BlockSpec((_BN, _C), lambda i, s: (0, 0)),
            pl.BlockSpec((1, _C), lambda i, s: (0, 0)),
            pl.BlockSpec((1, _BN), lambda i, s: (0, 0)),
        ],
        out_specs=[
            pl.BlockSpec((_TR, _BN), lambda i, s: (i, 0)),
            pl.BlockSpec((_TR, _BN), lambda i, s: (i, 0)),
        ],
        out_shape=(
            jax.ShapeDtypeStruct((_BN, _BN), jnp.float32),
            jax.ShapeDtypeStruct((_BN, _BN), jnp.float32),
        ),
        scratch_shapes=[pltpu.VMEM((_TR, _BN), jnp.float32)],
    )(fhall, f2, wch, eew)

    return (fout.reshape(_B, _N, _C), com, lat)
